# Initial kernel scaffold; baseline (speedup 1.0000x reference)
import jax
import jax.numpy as jnp
from jax.experimental import pallas as pl


def kernel(edge_index, x_init, W0, b0, W1, b1, W2, b2):
    raise NotImplementedError("write your pallas kernel here")


# speedup vs baseline: 5.1965x; 5.1965x over previous
import functools

import jax
import jax.numpy as jnp
from jax import lax
from jax.experimental import pallas as pl
from jax.experimental.pallas import tpu as pltpu
from jax.experimental.pallas import tpu_sc as plsc

N = 10000
E = 160000
IN_C = 256
HID = 512
OUT_C = 256

NC = 2
NS = 16
SLAB = 128
K = 80
EPT = E // NS
NCHUNK = EPT // K
NP = 10240
RPT = NP // NS
DEGW = 16


def _make_agg(n_slabs: int, with_deg: bool):
  spc = n_slabs // NC
  mesh = plsc.VectorSubcoreMesh(
      core_axis_name="c", subcore_axis_name="s", num_cores=NC, num_subcores=NS)

  out_type = [jax.ShapeDtypeStruct((NP, SLAB), jnp.float32)
              for _ in range(n_slabs)]
  if with_deg:
    out_type += [jax.ShapeDtypeStruct((NP, SLAB), jnp.float32)] * 2

  scratch = [
      pltpu.VMEM((K,), jnp.int32),
      pltpu.VMEM((K,), jnp.int32),
      pltpu.VMEM((K, SLAB), jnp.float32),
      pltpu.VMEM_SHARED((NP, SLAB), jnp.float32),
  ]
  if with_deg:
    scratch += [pltpu.VMEM((K, SLAB), jnp.float32)]

  def body(*refs):
    xs = refs[:n_slabs]
    row_h, col_h, zrow = refs[n_slabs:n_slabs + 3]
    i = n_slabs + 3
    if with_deg:
      ones_h = refs[i]; i += 1
    outs = refs[i:i + n_slabs]; i += n_slabs
    if with_deg:
      dega_out = refs[i]; degb_out = refs[i + 1]; i += 2
    row_v, col_v, rows_v, acc = refs[i:i + 4]; i += 4
    if with_deg:
      ones_v = refs[i]

    c = lax.axis_index("c")
    s = lax.axis_index("s")
    stripe = pl.ds(s * RPT, RPT)
    ebase = s * EPT

    if with_deg:
      pltpu.sync_copy(ones_h, ones_v)

    for p in range(spc):
      pltpu.sync_copy(zrow, acc.at[stripe])
      plsc.subcore_barrier()

      for core_id in range(NC):
        slab = p * NC + core_id
        x_s = xs[slab]

        @pl.when(c == core_id)
        def _():
          def chunk(j, carry):
            pltpu.sync_copy(row_h.at[pl.ds(ebase + j * K, K)], row_v)
            pltpu.sync_copy(col_h.at[pl.ds(ebase + j * K, K)], col_v)
            pltpu.sync_copy(x_s.at[row_v], rows_v)
            pltpu.sync_copy(rows_v, acc.at[col_v], add=True)
            return carry
          lax.fori_loop(0, NCHUNK, chunk, 0)
      plsc.subcore_barrier()

      for core_id in range(NC):
        out_s = outs[p * NC + core_id]

        @pl.when(c == core_id)
        def _():
          pltpu.sync_copy(acc.at[stripe], out_s.at[stripe])

    if with_deg:
      pltpu.sync_copy(zrow, acc.at[stripe])
      plsc.subcore_barrier()

      def dchunk(j, carry):
        pltpu.sync_copy(
            col_h.at[pl.ds(ebase + (2 * j + c) * K, K)], col_v)
        pltpu.sync_copy(ones_v, acc.at[col_v], add=True)
        return carry
      lax.fori_loop(0, NCHUNK // 2, dchunk, 0)

      @pl.when(c == 0)
      def _():
        pltpu.sync_copy(col_h.at[pl.ds(ebase + (NCHUNK - 1) * K, K)], col_v)
        pltpu.sync_copy(ones_v, acc.at[col_v], add=True)
      plsc.subcore_barrier()

      @pl.when(c == 0)
      def _():
        pltpu.sync_copy(acc.at[stripe], dega_out.at[stripe])

      @pl.when(c == 1)
      def _():
        pltpu.sync_copy(acc.at[stripe], degb_out.at[stripe])

  return functools.partial(
      pl.kernel, mesh=mesh, out_type=tuple(out_type),
      scratch_types=scratch)(body)


_agg2_deg = _make_agg(2, True)
_agg4 = _make_agg(4, False)
_agg2 = _make_agg(2, False)


NB = 1024


def _tc1_body(a0, a1, dega, degb, w0, b0, t0, t1, t2, t3):
  x = jnp.concatenate([a0[:], a1[:]], axis=1)
  h = jnp.dot(x, w0[:], preferred_element_type=jnp.float32) + b0[:]
  h = jnp.maximum(h, 0.0)
  dis = lax.rsqrt(dega[:, :1] + degb[:, :1] + 1.0)
  t = h * dis
  t0[:] = t[:, 0:128]
  t1[:] = t[:, 128:256]
  t2[:] = t[:, 256:384]
  t3[:] = t[:, 384:512]


def _tc1(a0, a1, dega, degb, w0, b0):
  slabspec = pl.BlockSpec((NB, SLAB), lambda i: (i, 0))
  return pl.pallas_call(
      _tc1_body,
      grid=(NP // NB,),
      in_specs=[slabspec, slabspec, slabspec, slabspec,
                pl.BlockSpec((IN_C, HID), lambda i: (0, 0)),
                pl.BlockSpec((1, HID), lambda i: (0, 0))],
      out_specs=[slabspec] * 4,
      out_shape=[jax.ShapeDtypeStruct((NP, SLAB), jnp.float32)] * 4,
  )(a0, a1, dega, degb, w0, b0)


def _tc2_body(s0, s1, s2, s3, t0, t1, t2, t3, dega, degb, w1, b1, w2, y0, y1):
  u = jnp.concatenate(
      [s0[:] + t0[:], s1[:] + t1[:], s2[:] + t2[:], s3[:] + t3[:]], axis=1)
  dis = lax.rsqrt(dega[:, :1] + degb[:, :1] + 1.0)
  u = u * dis
  h = jnp.dot(u, w1[:], preferred_element_type=jnp.float32) + b1[:]
  h = jnp.maximum(h, 0.0)
  y = jnp.dot(h, w2[:], preferred_element_type=jnp.float32)
  y0[:] = y[:, 0:128]
  y1[:] = y[:, 128:256]


def _tc2(s10, s11, s12, s13, t0, t1, t2, t3, dega, degb, w1, b1, w2):
  slabspec = pl.BlockSpec((NB, SLAB), lambda i: (i, 0))
  return pl.pallas_call(
      _tc2_body,
      grid=(NP // NB,),
      in_specs=[slabspec] * 10 + [
          pl.BlockSpec((HID, HID), lambda i: (0, 0)),
          pl.BlockSpec((1, HID), lambda i: (0, 0)),
          pl.BlockSpec((HID, OUT_C), lambda i: (0, 0))],
      out_specs=[slabspec] * 2,
      out_shape=[jax.ShapeDtypeStruct((NP, SLAB), jnp.float32)] * 2,
  )(s10, s11, s12, s13, t0, t1, t2, t3, dega, degb, w1, b1, w2)


def _tc3_body(s0, s1, b2, o):
  o[:] = jnp.concatenate([s0[:], s1[:]], axis=1) + b2[:]


def _tc3(s20, s21, b2):
  slabspec = pl.BlockSpec((NB, SLAB), lambda i: (i, 0))
  return pl.pallas_call(
      _tc3_body,
      grid=(NP // NB,),
      in_specs=[slabspec, slabspec,
                pl.BlockSpec((1, OUT_C), lambda i: (0, 0))],
      out_specs=pl.BlockSpec((NB, OUT_C), lambda i: (i, 0)),
      out_shape=jax.ShapeDtypeStruct((NP, OUT_C), jnp.float32),
  )(s20, s21, b2)


def kernel(edge_index, x_init, W0, b0, W1, b1, W2, b2):
  ei = edge_index.astype(jnp.int32)
  row = ei[0]
  col = ei[1]
  x0 = x_init[:, :SLAB]
  x1 = x_init[:, SLAB:]
  zrow = jnp.zeros((RPT, SLAB), jnp.float32)
  ones = jnp.ones((K, SLAB), jnp.float32)

  a0, a1, dega, degb = _agg2_deg(x0, x1, row, col, zrow, ones)
  t0, t1, t2, t3 = _tc1(a0, a1, dega, degb, W0, b0.reshape(1, HID))
  s10, s11, s12, s13 = _agg4(t0, t1, t2, t3, row, col, zrow)
  y0, y1 = _tc2(s10, s11, s12, s13, t0, t1, t2, t3, dega, degb,
                W1, b1.reshape(1, HID), W2)
  s20, s21 = _agg2(y0, y1, row, col, zrow)
  return _tc3(s20, s21, b2.reshape(1, OUT_C))[:N]


# speedup vs baseline: 11.9328x; 2.2963x over previous
import functools

import jax
import jax.numpy as jnp
from jax import lax
from jax.experimental import pallas as pl
from jax.experimental.pallas import tpu as pltpu
from jax.experimental.pallas import tpu_sc as plsc

N = 10000
E = 160000
IN_C = 256
HID = 512
OUT_C = 256

NC = 2
NS = 16
SLAB = 128
K = 80
EPT = E // NS
NCHUNK = EPT // K
NP = 10240
RPT = NP // NS
DEGW = 16


def _make_agg(n_slabs: int, with_deg: bool):
  spc = n_slabs // NC
  mesh = plsc.VectorSubcoreMesh(
      core_axis_name="c", subcore_axis_name="s", num_cores=NC, num_subcores=NS)

  out_type = [jax.ShapeDtypeStruct((NP, SLAB), jnp.float32)
              for _ in range(n_slabs)]
  if with_deg:
    out_type += [jax.ShapeDtypeStruct((NP, SLAB), jnp.float32)] * 2

  scratch = [
      pltpu.VMEM((EPT,), jnp.int32),
      pltpu.VMEM((EPT,), jnp.int32),
      pltpu.VMEM((K, SLAB), jnp.float32),
      pltpu.VMEM((K, SLAB), jnp.float32),
      pltpu.VMEM_SHARED((NP, SLAB), jnp.float32),
      pltpu.SemaphoreType.DMA,
      pltpu.SemaphoreType.DMA,
  ]

  def body(*refs):
    xs = refs[:n_slabs]
    row_h, col_h, zrow = refs[n_slabs:n_slabs + 3]
    i = n_slabs + 3
    if with_deg:
      ones_h = refs[i]; i += 1
    outs = refs[i:i + n_slabs]; i += n_slabs
    if with_deg:
      dega_out = refs[i]; degb_out = refs[i + 1]; i += 2
    row_v, col_v, buf0, buf1, acc, sem0, sem1 = refs[i:i + 7]; i += 7

    c = lax.axis_index("c")
    s = lax.axis_index("s")
    stripe = pl.ds(s * RPT, RPT)
    ebase = s * EPT

    pltpu.sync_copy(row_h.at[pl.ds(ebase, EPT)], row_v)
    pltpu.sync_copy(col_h.at[pl.ds(ebase, EPT)], col_v)

    def ridx(j):
      return row_v.at[pl.ds(j * K, K)]

    def cidx(j):
      return col_v.at[pl.ds(j * K, K)]

    for p in range(spc):
      pltpu.sync_copy(zrow, acc.at[stripe])
      plsc.subcore_barrier()

      for core_id in range(NC):
        slab = p * NC + core_id
        x_s = xs[slab]

        @pl.when(c == core_id)
        def _():
          pltpu.async_copy(x_s.at[ridx(0)], buf0, sem0)

          def chunk(i2, carry):
            j = 2 * i2
            pltpu.async_copy(x_s.at[ridx(j + 1)], buf1, sem1)
            pltpu.make_async_copy(x_s.at[ridx(j)], buf0, sem0).wait()
            pltpu.sync_copy(buf0, acc.at[cidx(j)], add=True)
            pltpu.async_copy(x_s.at[ridx(j + 2)], buf0, sem0)
            pltpu.make_async_copy(x_s.at[ridx(j + 1)], buf1, sem1).wait()
            pltpu.sync_copy(buf1, acc.at[cidx(j + 1)], add=True)
            return carry
          lax.fori_loop(0, NCHUNK // 2, chunk, 0)
          pltpu.make_async_copy(x_s.at[ridx(NCHUNK - 1)], buf0, sem0).wait()
          pltpu.sync_copy(buf0, acc.at[cidx(NCHUNK - 1)], add=True)
      plsc.subcore_barrier()

      for core_id in range(NC):
        out_s = outs[p * NC + core_id]

        @pl.when(c == core_id)
        def _():
          pltpu.sync_copy(acc.at[stripe], out_s.at[stripe])

    if with_deg:
      pltpu.sync_copy(zrow, acc.at[stripe])
      pltpu.sync_copy(ones_h, buf0)
      plsc.subcore_barrier()

      def dchunk(j, carry):
        pltpu.sync_copy(buf0, acc.at[cidx(2 * j + c)], add=True)
        return carry
      lax.fori_loop(0, NCHUNK // 2, dchunk, 0)

      @pl.when(c == 0)
      def _():
        pltpu.sync_copy(buf0, acc.at[cidx(NCHUNK - 1)], add=True)
      plsc.subcore_barrier()

      @pl.when(c == 0)
      def _():
        pltpu.sync_copy(acc.at[stripe], dega_out.at[stripe])

      @pl.when(c == 1)
      def _():
        pltpu.sync_copy(acc.at[stripe], degb_out.at[stripe])

  return functools.partial(
      pl.kernel, mesh=mesh, out_type=tuple(out_type),
      scratch_types=scratch)(body)


_agg2_deg = _make_agg(2, True)
_agg4 = _make_agg(4, False)
_agg2 = _make_agg(2, False)


NB = 1024


def _tc1_body(a0, a1, dega, degb, w0, b0, t0, t1, t2, t3):
  x = jnp.concatenate([a0[:], a1[:]], axis=1)
  h = jnp.dot(x, w0[:], preferred_element_type=jnp.float32) + b0[:]
  h = jnp.maximum(h, 0.0)
  dis = lax.rsqrt(dega[:, :1] + degb[:, :1] + 1.0)
  t = h * dis
  t0[:] = t[:, 0:128]
  t1[:] = t[:, 128:256]
  t2[:] = t[:, 256:384]
  t3[:] = t[:, 384:512]


def _tc1(a0, a1, dega, degb, w0, b0):
  slabspec = pl.BlockSpec((NB, SLAB), lambda i: (i, 0))
  return pl.pallas_call(
      _tc1_body,
      grid=(NP // NB,),
      in_specs=[slabspec, slabspec, slabspec, slabspec,
                pl.BlockSpec((IN_C, HID), lambda i: (0, 0)),
                pl.BlockSpec((1, HID), lambda i: (0, 0))],
      out_specs=[slabspec] * 4,
      out_shape=[jax.ShapeDtypeStruct((NP, SLAB), jnp.float32)] * 4,
  )(a0, a1, dega, degb, w0, b0)


def _tc2_body(s0, s1, s2, s3, t0, t1, t2, t3, dega, degb, w1, b1, w2, y0, y1):
  u = jnp.concatenate(
      [s0[:] + t0[:], s1[:] + t1[:], s2[:] + t2[:], s3[:] + t3[:]], axis=1)
  dis = lax.rsqrt(dega[:, :1] + degb[:, :1] + 1.0)
  u = u * dis
  h = jnp.dot(u, w1[:], preferred_element_type=jnp.float32) + b1[:]
  h = jnp.maximum(h, 0.0)
  y = jnp.dot(h, w2[:], preferred_element_type=jnp.float32)
  y0[:] = y[:, 0:128]
  y1[:] = y[:, 128:256]


def _tc2(s10, s11, s12, s13, t0, t1, t2, t3, dega, degb, w1, b1, w2):
  slabspec = pl.BlockSpec((NB, SLAB), lambda i: (i, 0))
  return pl.pallas_call(
      _tc2_body,
      grid=(NP // NB,),
      in_specs=[slabspec] * 10 + [
          pl.BlockSpec((HID, HID), lambda i: (0, 0)),
          pl.BlockSpec((1, HID), lambda i: (0, 0)),
          pl.BlockSpec((HID, OUT_C), lambda i: (0, 0))],
      out_specs=[slabspec] * 2,
      out_shape=[jax.ShapeDtypeStruct((NP, SLAB), jnp.float32)] * 2,
  )(s10, s11, s12, s13, t0, t1, t2, t3, dega, degb, w1, b1, w2)


def _tc3_body(s0, s1, b2, o):
  o[:] = jnp.concatenate([s0[:], s1[:]], axis=1) + b2[:]


def _tc3(s20, s21, b2):
  slabspec = pl.BlockSpec((NB, SLAB), lambda i: (i, 0))
  return pl.pallas_call(
      _tc3_body,
      grid=(NP // NB,),
      in_specs=[slabspec, slabspec,
                pl.BlockSpec((1, OUT_C), lambda i: (0, 0))],
      out_specs=pl.BlockSpec((NB, OUT_C), lambda i: (i, 0)),
      out_shape=jax.ShapeDtypeStruct((NP, OUT_C), jnp.float32),
  )(s20, s21, b2)


def kernel(edge_index, x_init, W0, b0, W1, b1, W2, b2):
  ei = edge_index.astype(jnp.int32)
  row = ei[0]
  col = ei[1]
  x0 = x_init[:, :SLAB]
  x1 = x_init[:, SLAB:]
  zrow = jnp.zeros((RPT, SLAB), jnp.float32)
  ones = jnp.ones((K, SLAB), jnp.float32)

  a0, a1, dega, degb = _agg2_deg(x0, x1, row, col, zrow, ones)
  t0, t1, t2, t3 = _tc1(a0, a1, dega, degb, W0, b0.reshape(1, HID))
  s10, s11, s12, s13 = _agg4(t0, t1, t2, t3, row, col, zrow)
  y0, y1 = _tc2(s10, s11, s12, s13, t0, t1, t2, t3, dega, degb,
                W1, b1.reshape(1, HID), W2)
  s20, s21 = _agg2(y0, y1, row, col, zrow)
  return _tc3(s20, s21, b2.reshape(1, OUT_C))[:N]
